# initial kernel scaffold (unmeasured)
import jax
import jax.numpy as jnp
from jax import lax
from jax.experimental import pallas as pl
from jax.experimental.pallas import tpu as pltpu


def kernel(
    x,
):
    def body(*refs):
        pass

    out_shape = jax.ShapeDtypeStruct(..., jnp.float32)
    return pl.pallas_call(body, out_shape=out_shape)(...)


# baseline (device time: 400337 ns/iter reference)
import jax
import jax.numpy as jnp
from jax import lax
from jax.experimental import pallas as pl
from jax.experimental.pallas import tpu as pltpu

M, N = 4096, 2048


def kernel(x):
    xb = x.reshape(M, N).astype(jnp.bfloat16)

    def body(x_ref, out_ref, recv_ref, send_sems, recv_sems, ready_sem):
        my_x = lax.axis_index("x")
        my_y = lax.axis_index("y")
        x_nbr = (1 - my_x, my_y)
        y_nbr = (my_x, 1 - my_y)

        barrier_sem = pltpu.get_barrier_semaphore()
        for nbr in (x_nbr, y_nbr):
            pl.semaphore_signal(
                barrier_sem, inc=1,
                device_id=nbr, device_id_type=pl.DeviceIdType.MESH,
            )
        pl.semaphore_wait(barrier_sem, 2)

        out_ref[...] = x_ref[...]

        rdma_x = pltpu.make_async_remote_copy(
            src_ref=out_ref,
            dst_ref=recv_ref,
            send_sem=send_sems.at[0],
            recv_sem=recv_sems.at[0],
            device_id=x_nbr,
            device_id_type=pl.DeviceIdType.MESH,
        )
        rdma_x.start()
        rdma_x.wait()
        out_ref[...] = out_ref[...] + recv_ref[...]

        pl.semaphore_signal(
            ready_sem, inc=1,
            device_id=y_nbr, device_id_type=pl.DeviceIdType.MESH,
        )
        pl.semaphore_wait(ready_sem, 1)

        rdma_y = pltpu.make_async_remote_copy(
            src_ref=out_ref,
            dst_ref=recv_ref,
            send_sem=send_sems.at[1],
            recv_sem=recv_sems.at[1],
            device_id=y_nbr,
            device_id_type=pl.DeviceIdType.MESH,
        )
        rdma_y.start()
        rdma_y.wait()
        out_ref[...] = out_ref[...] + recv_ref[...]

    return pl.pallas_call(
        body,
        out_shape=jax.ShapeDtypeStruct((M, N), jnp.bfloat16),
        in_specs=[pl.BlockSpec(memory_space=pltpu.VMEM)],
        out_specs=pl.BlockSpec(memory_space=pltpu.VMEM),
        scratch_shapes=[
            pltpu.VMEM((M, N), jnp.bfloat16),
            pltpu.SemaphoreType.DMA((2,)),
            pltpu.SemaphoreType.DMA((2,)),
            pltpu.SemaphoreType.REGULAR,
        ],
        compiler_params=pltpu.CompilerParams(collective_id=0),
    )(xb)


# device time: 174473 ns/iter; 2.2945x vs baseline; 2.2945x over previous
import jax
import jax.numpy as jnp
from jax import lax
from jax.experimental import pallas as pl
from jax.experimental.pallas import tpu as pltpu

M, N = 4096, 2048
H = M // 2
C = H // 2
S = C // 2


def kernel(x):
    xb = x.reshape(M, N).astype(jnp.bfloat16)

    def body(x_ref, out_ref, recv_ref, send_sems, recv_sems):
        mx = lax.axis_index("x")
        my = lax.axis_index("y")
        x_nbr = (1 - mx, my)
        y_nbr = (mx, 1 - my)

        barrier_sem = pltpu.get_barrier_semaphore()
        for nbr in (x_nbr, y_nbr):
            pl.semaphore_signal(
                barrier_sem, inc=1,
                device_id=nbr, device_id_type=pl.DeviceIdType.MESH,
            )
        pl.semaphore_wait(barrier_sem, 2)

        def copy(src, dst, i, dev):
            return pltpu.make_async_remote_copy(
                src_ref=src, dst_ref=dst,
                send_sem=send_sems.at[i], recv_sem=recv_sems.at[i],
                device_id=dev, device_id_type=pl.DeviceIdType.MESH,
            )

        a_chunk = mx * C
        a_sub = a_chunk + my * S
        b_chunk = H + my * C
        b_sub = b_chunk + mx * S

        A1 = copy(x_ref.at[pl.ds((1 - mx) * C, C), :],
                  recv_ref.at[pl.ds(0, C), :], 0, x_nbr)
        B1 = copy(x_ref.at[pl.ds(H + (1 - my) * C, C), :],
                  recv_ref.at[pl.ds(C, C), :], 1, y_nbr)
        A1.start()
        B1.start()

        A1.wait_recv()
        out_ref[pl.ds(a_chunk, C), :] = (
            x_ref[pl.ds(a_chunk, C), :] + recv_ref[pl.ds(0, C), :])
        A2 = copy(out_ref.at[pl.ds(a_chunk + (1 - my) * S, S), :],
                  recv_ref.at[pl.ds(2 * C, S), :], 2, y_nbr)
        A2.start()

        B1.wait_recv()
        out_ref[pl.ds(b_chunk, C), :] = (
            x_ref[pl.ds(b_chunk, C), :] + recv_ref[pl.ds(C, C), :])
        B2 = copy(out_ref.at[pl.ds(b_chunk + (1 - mx) * S, S), :],
                  recv_ref.at[pl.ds(2 * C + S, S), :], 3, x_nbr)
        B2.start()

        A2.wait_recv()
        out_ref[pl.ds(a_sub, S), :] = (
            out_ref[pl.ds(a_sub, S), :] + recv_ref[pl.ds(2 * C, S), :])
        A3s = copy(out_ref.at[pl.ds(a_sub, S), :],
                   out_ref.at[pl.ds(a_sub, S), :], 4, y_nbr)
        A3s.start()

        B2.wait_recv()
        out_ref[pl.ds(b_sub, S), :] = (
            out_ref[pl.ds(b_sub, S), :] + recv_ref[pl.ds(2 * C + S, S), :])
        B3s = copy(out_ref.at[pl.ds(b_sub, S), :],
                   out_ref.at[pl.ds(b_sub, S), :], 5, x_nbr)
        B3s.start()

        A3r = copy(out_ref.at[pl.ds(a_chunk + (1 - my) * S, S), :],
                   out_ref.at[pl.ds(a_chunk + (1 - my) * S, S), :], 4, y_nbr)
        A3r.wait_recv()
        A4s = copy(out_ref.at[pl.ds(a_chunk, C), :],
                   out_ref.at[pl.ds(a_chunk, C), :], 6, x_nbr)
        A4s.start()

        B3r = copy(out_ref.at[pl.ds(b_chunk + (1 - mx) * S, S), :],
                   out_ref.at[pl.ds(b_chunk + (1 - mx) * S, S), :], 5, x_nbr)
        B3r.wait_recv()
        B4s = copy(out_ref.at[pl.ds(b_chunk, C), :],
                   out_ref.at[pl.ds(b_chunk, C), :], 7, y_nbr)
        B4s.start()

        A4r = copy(out_ref.at[pl.ds((1 - mx) * C, C), :],
                   out_ref.at[pl.ds((1 - mx) * C, C), :], 6, x_nbr)
        A4r.wait_recv()
        B4r = copy(out_ref.at[pl.ds(H + (1 - my) * C, C), :],
                   out_ref.at[pl.ds(H + (1 - my) * C, C), :], 7, y_nbr)
        B4r.wait_recv()

        A1.wait_send()
        B1.wait_send()
        A2.wait_send()
        B2.wait_send()
        A3s.wait_send()
        B3s.wait_send()
        A4s.wait_send()
        B4s.wait_send()

    return pl.pallas_call(
        body,
        out_shape=jax.ShapeDtypeStruct((M, N), jnp.bfloat16),
        in_specs=[pl.BlockSpec(memory_space=pltpu.VMEM)],
        out_specs=pl.BlockSpec(memory_space=pltpu.VMEM),
        scratch_shapes=[
            pltpu.VMEM((2 * C + 2 * S, N), jnp.bfloat16),
            pltpu.SemaphoreType.DMA((8,)),
            pltpu.SemaphoreType.DMA((8,)),
        ],
        compiler_params=pltpu.CompilerParams(collective_id=0),
    )(xb)


# device time: 172395 ns/iter; 2.3222x vs baseline; 1.0121x over previous
import jax
import jax.numpy as jnp
from jax import lax
from jax.experimental import pallas as pl
from jax.experimental.pallas import tpu as pltpu

M, N = 4096, 2048
H = M // 2
C = H // 2
S = C // 2


def kernel(x):
    xb = x.reshape(M, N).astype(jnp.bfloat16)

    def body(x_ref, out_ref, recv_ref, send_sems, recv_sems):
        mx = lax.axis_index("x")
        my = lax.axis_index("y")
        x_nbr = (1 - mx, my)
        y_nbr = (mx, 1 - my)

        barrier_sem = pltpu.get_barrier_semaphore()
        for nbr in (x_nbr, y_nbr):
            pl.semaphore_signal(
                barrier_sem, inc=1,
                device_id=nbr, device_id_type=pl.DeviceIdType.MESH,
            )
        pl.semaphore_wait(barrier_sem, 2)

        def copy(src, dst, i, dev):
            return pltpu.make_async_remote_copy(
                src_ref=src, dst_ref=dst,
                send_sem=send_sems.at[i], recv_sem=recv_sems.at[i],
                device_id=dev, device_id_type=pl.DeviceIdType.MESH,
            )

        a_chunk = mx * C
        a_sub = a_chunk + my * S
        a_osub = a_chunk + (1 - my) * S
        b_chunk = H + my * C
        b_sub = b_chunk + mx * S
        b_osub = b_chunk + (1 - mx) * S

        A1a = copy(x_ref.at[pl.ds((1 - mx) * C + (1 - my) * S, S), :],
                   recv_ref.at[pl.ds((1 - my) * S, S), :], 0, x_nbr)
        A1b = copy(x_ref.at[pl.ds((1 - mx) * C + my * S, S), :],
                   recv_ref.at[pl.ds(my * S, S), :], 1, x_nbr)
        B1a = copy(x_ref.at[pl.ds(H + (1 - my) * C + (1 - mx) * S, S), :],
                   recv_ref.at[pl.ds(C + (1 - mx) * S, S), :], 2, y_nbr)
        B1b = copy(x_ref.at[pl.ds(H + (1 - my) * C + mx * S, S), :],
                   recv_ref.at[pl.ds(C + mx * S, S), :], 3, y_nbr)
        A1a.start()
        A1b.start()
        B1a.start()
        B1b.start()

        A1a.wait_recv()
        out_ref[pl.ds(a_osub, S), :] = (
            x_ref[pl.ds(a_osub, S), :]
            + recv_ref[pl.ds((1 - my) * S, S), :])
        A2 = copy(out_ref.at[pl.ds(a_osub, S), :],
                  recv_ref.at[pl.ds(2 * C, S), :], 4, y_nbr)
        A2.start()

        B1a.wait_recv()
        out_ref[pl.ds(b_osub, S), :] = (
            x_ref[pl.ds(b_osub, S), :]
            + recv_ref[pl.ds(C + (1 - mx) * S, S), :])
        B2 = copy(out_ref.at[pl.ds(b_osub, S), :],
                  recv_ref.at[pl.ds(2 * C + S, S), :], 5, x_nbr)
        B2.start()

        A1b.wait_recv()
        out_ref[pl.ds(a_sub, S), :] = (
            x_ref[pl.ds(a_sub, S), :] + recv_ref[pl.ds(my * S, S), :])
        B1b.wait_recv()
        out_ref[pl.ds(b_sub, S), :] = (
            x_ref[pl.ds(b_sub, S), :] + recv_ref[pl.ds(C + mx * S, S), :])

        A2.wait_recv()
        out_ref[pl.ds(a_sub, S), :] = (
            out_ref[pl.ds(a_sub, S), :] + recv_ref[pl.ds(2 * C, S), :])
        A3s = copy(out_ref.at[pl.ds(a_sub, S), :],
                   out_ref.at[pl.ds(a_sub, S), :], 6, y_nbr)
        A3s.start()
        A4a = copy(out_ref.at[pl.ds(a_sub, S), :],
                   out_ref.at[pl.ds(a_sub, S), :], 8, x_nbr)
        A4a.start()

        B2.wait_recv()
        out_ref[pl.ds(b_sub, S), :] = (
            out_ref[pl.ds(b_sub, S), :] + recv_ref[pl.ds(2 * C + S, S), :])
        B3s = copy(out_ref.at[pl.ds(b_sub, S), :],
                   out_ref.at[pl.ds(b_sub, S), :], 7, x_nbr)
        B3s.start()
        B4a = copy(out_ref.at[pl.ds(b_sub, S), :],
                   out_ref.at[pl.ds(b_sub, S), :], 10, y_nbr)
        B4a.start()

        A3r = copy(out_ref.at[pl.ds(a_osub, S), :],
                   out_ref.at[pl.ds(a_osub, S), :], 6, y_nbr)
        A3r.wait_recv()
        A4b = copy(out_ref.at[pl.ds(a_osub, S), :],
                   out_ref.at[pl.ds(a_osub, S), :], 9, x_nbr)
        A4b.start()

        B3r = copy(out_ref.at[pl.ds(b_osub, S), :],
                   out_ref.at[pl.ds(b_osub, S), :], 7, x_nbr)
        B3r.wait_recv()
        B4b = copy(out_ref.at[pl.ds(b_osub, S), :],
                   out_ref.at[pl.ds(b_osub, S), :], 11, y_nbr)
        B4b.start()

        A4ar = copy(out_ref.at[pl.ds((1 - mx) * C + my * S, S), :],
                    out_ref.at[pl.ds((1 - mx) * C + my * S, S), :], 8, x_nbr)
        A4ar.wait_recv()
        A4br = copy(out_ref.at[pl.ds((1 - mx) * C + (1 - my) * S, S), :],
                    out_ref.at[pl.ds((1 - mx) * C + (1 - my) * S, S), :],
                    9, x_nbr)
        A4br.wait_recv()
        B4ar = copy(out_ref.at[pl.ds(H + (1 - my) * C + mx * S, S), :],
                    out_ref.at[pl.ds(H + (1 - my) * C + mx * S, S), :],
                    10, y_nbr)
        B4ar.wait_recv()
        B4br = copy(out_ref.at[pl.ds(H + (1 - my) * C + (1 - mx) * S, S), :],
                    out_ref.at[pl.ds(H + (1 - my) * C + (1 - mx) * S, S), :],
                    11, y_nbr)
        B4br.wait_recv()

        for s in (A1a, A1b, B1a, B1b, A2, B2, A3s, B3s, A4a, A4b, B4a, B4b):
            s.wait_send()

    return pl.pallas_call(
        body,
        out_shape=jax.ShapeDtypeStruct((M, N), jnp.bfloat16),
        in_specs=[pl.BlockSpec(memory_space=pltpu.VMEM)],
        out_specs=pl.BlockSpec(memory_space=pltpu.VMEM),
        scratch_shapes=[
            pltpu.VMEM((2 * C + 2 * S, N), jnp.bfloat16),
            pltpu.SemaphoreType.DMA((12,)),
            pltpu.SemaphoreType.DMA((12,)),
        ],
        compiler_params=pltpu.CompilerParams(collective_id=0),
    )(xb)


# device time: 165306 ns/iter; 2.4218x vs baseline; 1.0429x over previous
import jax
import jax.numpy as jnp
from jax import lax
from jax.experimental import pallas as pl
from jax.experimental.pallas import tpu as pltpu

M, N = 4096, 2048
H = M // 2
C = H // 2
S = C // 2


def kernel(x):
    xq = x.reshape(M, N)

    def body(x_hbm, out_ref, xb, stage, recv_ref,
             stage_sems, send_sems, recv_sems):
        mx = lax.axis_index("x")
        my = lax.axis_index("y")
        x_nbr = (1 - mx, my)
        y_nbr = (mx, 1 - my)

        a_chunk = mx * C
        a_sub = a_chunk + my * S
        a_osub = a_chunk + (1 - my) * S
        b_chunk = H + my * C
        b_sub = b_chunk + mx * S
        b_osub = b_chunk + (1 - mx) * S

        rows = [
            (1 - mx) * C + (1 - my) * S,
            H + (1 - my) * C + (1 - mx) * S,
            (1 - mx) * C + my * S,
            H + (1 - my) * C + mx * S,
            a_osub,
            b_osub,
            a_sub,
            b_sub,
        ]

        dmas = {}
        for k in (0, 1):
            dmas[k] = pltpu.make_async_copy(
                x_hbm.at[pl.ds(rows[k], S), :],
                stage.at[k % 2], stage_sems.at[k % 2])
            dmas[k].start()

        barrier_sem = pltpu.get_barrier_semaphore()
        for nbr in (x_nbr, y_nbr):
            pl.semaphore_signal(
                barrier_sem, inc=1,
                device_id=nbr, device_id_type=pl.DeviceIdType.MESH,
            )
        pl.semaphore_wait(barrier_sem, 2)

        def copy(src, dst, i, dev):
            return pltpu.make_async_remote_copy(
                src_ref=src, dst_ref=dst,
                send_sem=send_sems.at[i], recv_sem=recv_sems.at[i],
                device_id=dev, device_id_type=pl.DeviceIdType.MESH,
            )

        A1a = copy(xb.at[pl.ds(rows[0], S), :],
                   recv_ref.at[pl.ds((1 - my) * S, S), :], 0, x_nbr)
        B1a = copy(xb.at[pl.ds(rows[1], S), :],
                   recv_ref.at[pl.ds(C + (1 - mx) * S, S), :], 2, y_nbr)
        A1b = copy(xb.at[pl.ds(rows[2], S), :],
                   recv_ref.at[pl.ds(my * S, S), :], 1, x_nbr)
        B1b = copy(xb.at[pl.ds(rows[3], S), :],
                   recv_ref.at[pl.ds(C + mx * S, S), :], 3, y_nbr)
        sends = {0: A1a, 1: B1a, 2: A1b, 3: B1b}

        for k in range(8):
            dmas[k].wait()
            xb[pl.ds(rows[k], S), :] = stage[k % 2].astype(jnp.bfloat16)
            if k in sends:
                sends[k].start()
            if k + 2 < 8:
                dmas[k + 2] = pltpu.make_async_copy(
                    x_hbm.at[pl.ds(rows[k + 2], S), :],
                    stage.at[k % 2], stage_sems.at[k % 2])
                dmas[k + 2].start()

        A1a.wait_recv()
        out_ref[pl.ds(a_osub, S), :] = (
            xb[pl.ds(a_osub, S), :]
            + recv_ref[pl.ds((1 - my) * S, S), :])
        A2 = copy(out_ref.at[pl.ds(a_osub, S), :],
                  recv_ref.at[pl.ds(2 * C, S), :], 4, y_nbr)
        A2.start()

        B1a.wait_recv()
        out_ref[pl.ds(b_osub, S), :] = (
            xb[pl.ds(b_osub, S), :]
            + recv_ref[pl.ds(C + (1 - mx) * S, S), :])
        B2 = copy(out_ref.at[pl.ds(b_osub, S), :],
                  recv_ref.at[pl.ds(2 * C + S, S), :], 5, x_nbr)
        B2.start()

        A1b.wait_recv()
        out_ref[pl.ds(a_sub, S), :] = (
            xb[pl.ds(a_sub, S), :] + recv_ref[pl.ds(my * S, S), :])
        B1b.wait_recv()
        out_ref[pl.ds(b_sub, S), :] = (
            xb[pl.ds(b_sub, S), :] + recv_ref[pl.ds(C + mx * S, S), :])

        A2.wait_recv()
        out_ref[pl.ds(a_sub, S), :] = (
            out_ref[pl.ds(a_sub, S), :] + recv_ref[pl.ds(2 * C, S), :])
        A3s = copy(out_ref.at[pl.ds(a_sub, S), :],
                   out_ref.at[pl.ds(a_sub, S), :], 6, y_nbr)
        A3s.start()
        A4a = copy(out_ref.at[pl.ds(a_sub, S), :],
                   out_ref.at[pl.ds(a_sub, S), :], 8, x_nbr)
        A4a.start()

        B2.wait_recv()
        out_ref[pl.ds(b_sub, S), :] = (
            out_ref[pl.ds(b_sub, S), :] + recv_ref[pl.ds(2 * C + S, S), :])
        B3s = copy(out_ref.at[pl.ds(b_sub, S), :],
                   out_ref.at[pl.ds(b_sub, S), :], 7, x_nbr)
        B3s.start()
        B4a = copy(out_ref.at[pl.ds(b_sub, S), :],
                   out_ref.at[pl.ds(b_sub, S), :], 10, y_nbr)
        B4a.start()

        A3r = copy(out_ref.at[pl.ds(a_osub, S), :],
                   out_ref.at[pl.ds(a_osub, S), :], 6, y_nbr)
        A3r.wait_recv()
        A4b = copy(out_ref.at[pl.ds(a_osub, S), :],
                   out_ref.at[pl.ds(a_osub, S), :], 9, x_nbr)
        A4b.start()

        B3r = copy(out_ref.at[pl.ds(b_osub, S), :],
                   out_ref.at[pl.ds(b_osub, S), :], 7, x_nbr)
        B3r.wait_recv()
        B4b = copy(out_ref.at[pl.ds(b_osub, S), :],
                   out_ref.at[pl.ds(b_osub, S), :], 11, y_nbr)
        B4b.start()

        A4ar = copy(out_ref.at[pl.ds((1 - mx) * C + my * S, S), :],
                    out_ref.at[pl.ds((1 - mx) * C + my * S, S), :], 8, x_nbr)
        A4ar.wait_recv()
        A4br = copy(out_ref.at[pl.ds((1 - mx) * C + (1 - my) * S, S), :],
                    out_ref.at[pl.ds((1 - mx) * C + (1 - my) * S, S), :],
                    9, x_nbr)
        A4br.wait_recv()
        B4ar = copy(out_ref.at[pl.ds(H + (1 - my) * C + mx * S, S), :],
                    out_ref.at[pl.ds(H + (1 - my) * C + mx * S, S), :],
                    10, y_nbr)
        B4ar.wait_recv()
        B4br = copy(out_ref.at[pl.ds(H + (1 - my) * C + (1 - mx) * S, S), :],
                    out_ref.at[pl.ds(H + (1 - my) * C + (1 - mx) * S, S), :],
                    11, y_nbr)
        B4br.wait_recv()

        for s in (A1a, A1b, B1a, B1b, A2, B2, A3s, B3s, A4a, A4b, B4a, B4b):
            s.wait_send()

    return pl.pallas_call(
        body,
        out_shape=jax.ShapeDtypeStruct((M, N), jnp.bfloat16),
        in_specs=[pl.BlockSpec(memory_space=pltpu.MemorySpace.HBM)],
        out_specs=pl.BlockSpec(memory_space=pltpu.VMEM),
        scratch_shapes=[
            pltpu.VMEM((M, N), jnp.bfloat16),
            pltpu.VMEM((2, S, N), jnp.float32),
            pltpu.VMEM((2 * C + 2 * S, N), jnp.bfloat16),
            pltpu.SemaphoreType.DMA((2,)),
            pltpu.SemaphoreType.DMA((12,)),
            pltpu.SemaphoreType.DMA((12,)),
        ],
        compiler_params=pltpu.CompilerParams(
            collective_id=0, vmem_limit_bytes=56 * 1024 * 1024,
        ),
    )(xq)
